# baseline (speedup 1.0000x reference)
import functools

import jax
import jax.numpy as jnp
from jax import lax
from jax.experimental import pallas as pl
from jax.experimental.pallas import tpu as pltpu
from jax.experimental.pallas import tpu_sc as plsc

_R, _C = 1_000_000, 64
_H = 16

_mesh = plsc.VectorSubcoreMesh(core_axis_name="c", subcore_axis_name="s")


@functools.partial(
    pl.kernel,
    out_type=jax.ShapeDtypeStruct((_H, _C), jnp.float32),
    mesh=_mesh,
    scratch_types=[pltpu.VMEM((_H, _C), jnp.float32)],
)
def _sc_scatter_head(a_hbm, o_hbm, buf):
    wid = lax.axis_index("s") * 2 + lax.axis_index("c")

    @pl.when(wid == 0)
    def _():
        pltpu.sync_copy(a_hbm, buf)
        lane = lax.iota(jnp.int32, 16)
        buf[0, pl.ds(0, 16)] = buf[0, pl.ds(0, 16)] + jnp.where(
            lane == 0, jnp.float32(1.0), jnp.float32(0.0)
        )
        one_at_lane0 = jnp.where(lane == 0, jnp.float32(1.0), jnp.float32(0.0))
        buf[1, pl.ds(0, 16)] = buf[1, pl.ds(0, 16)] + one_at_lane0
        buf[1, pl.ds(0, 16)] = buf[1, pl.ds(0, 16)] + one_at_lane0
        pltpu.sync_copy(buf, o_hbm)


def kernel(A):
    head = _sc_scatter_head(A[:_H])
    bulk = jax.lax.dynamic_update_slice(A, A[:_H], (0, 0))
    return jax.lax.dynamic_update_slice(bulk, head, (0, 0))


# speedup vs baseline: 1.0090x; 1.0090x over previous
import functools

import jax
import jax.numpy as jnp
from jax import lax
from jax.experimental import pallas as pl
from jax.experimental.pallas import tpu as pltpu
from jax.experimental.pallas import tpu_sc as plsc

_R, _C = 1_000_000, 64
_H = 16

_mesh = plsc.VectorSubcoreMesh(core_axis_name="c", subcore_axis_name="s", num_cores=1)


@functools.partial(
    pl.kernel,
    out_type=jax.ShapeDtypeStruct((_H, _C), jnp.float32),
    mesh=_mesh,
    scratch_types=[pltpu.VMEM((_H, _C), jnp.float32)],
)
def _sc_scatter_head(a_hbm, o_hbm, buf):
    wid = lax.axis_index("s") + lax.axis_index("c")

    @pl.when(wid == 0)
    def _():
        pltpu.sync_copy(a_hbm, buf)
        lane = lax.iota(jnp.int32, 16)
        buf[0, pl.ds(0, 16)] = buf[0, pl.ds(0, 16)] + jnp.where(
            lane == 0, jnp.float32(1.0), jnp.float32(0.0)
        )
        one_at_lane0 = jnp.where(lane == 0, jnp.float32(1.0), jnp.float32(0.0))
        buf[1, pl.ds(0, 16)] = buf[1, pl.ds(0, 16)] + one_at_lane0
        buf[1, pl.ds(0, 16)] = buf[1, pl.ds(0, 16)] + one_at_lane0
        pltpu.sync_copy(buf, o_hbm)


def kernel(A):
    head = _sc_scatter_head(A[:_H])
    return jax.lax.dynamic_update_slice(A, head, (0, 0))


# speedup vs baseline: 1.0121x; 1.0031x over previous
import functools
import jax
import jax.numpy as jnp
from jax import lax
from jax.experimental import pallas as pl
from jax.experimental.pallas import tpu as pltpu
from jax.experimental.pallas import tpu_sc as plsc

_R, _C = 1_000_000, 64
_H = 16

_mesh = plsc.ScalarSubcoreMesh(axis_name="c", num_cores=1)


@functools.partial(
    pl.kernel,
    out_type=jax.ShapeDtypeStruct((_H, _C), jnp.float32),
    mesh=_mesh,
    scratch_types=[pltpu.SMEM((_H, _C), jnp.float32)],
)
def _sc_scatter_head(a_hbm, o_hbm, buf):
    @pl.when(lax.axis_index("c") == 0)
    def _():
        pltpu.sync_copy(a_hbm, buf)
        buf[0, 0] = buf[0, 0] + jnp.float32(1.0)
        buf[1, 0] = buf[1, 0] + jnp.float32(1.0)
        buf[1, 0] = buf[1, 0] + jnp.float32(1.0)
        pltpu.sync_copy(buf, o_hbm)


def kernel(A):
    head = _sc_scatter_head(A[:_H])
    return jax.lax.dynamic_update_slice(A, head, (0, 0))
